# Initial kernel scaffold; baseline (speedup 1.0000x reference)
import jax
import jax.numpy as jnp
from jax.experimental import pallas as pl


def kernel(x, W1, a_src1, a_dst1, b1, W2, a_src2, a_dst2, b2, emb, conv_w, conv_b, fcg1_w, fcg1_b, fcxt1_w, fcxt1_b, fc1_w, fc1_b, fc2_w, fc2_b, out_w, out_b, edge_index, batch, target):
    raise NotImplementedError("write your pallas kernel here")


# speedup vs baseline: 1.0723x; 1.0723x over previous
import jax
import jax.numpy as jnp
from jax.experimental import pallas as pl
from jax.experimental.pallas import tpu as pltpu


def _head_body(xc_ref, w1_ref, b1_ref, w2_ref, b2_ref, wo_ref, bo_ref, out_ref):
    xc = xc_ref[...]
    h = jnp.maximum(jnp.dot(xc, w1_ref[...], preferred_element_type=jnp.float32) + b1_ref[...], 0.0)
    h = jnp.maximum(jnp.dot(h, w2_ref[...], preferred_element_type=jnp.float32) + b2_ref[...], 0.0)
    out_ref[...] = jnp.dot(h, wo_ref[...], preferred_element_type=jnp.float32) + bo_ref[...]


def _mlp_head(xc, fc1_w, fc1_b, fc2_w, fc2_b, out_w, out_b):
    B = xc.shape[0]
    return pl.pallas_call(
        _head_body,
        out_shape=jax.ShapeDtypeStruct((B, 1), jnp.float32),
    )(xc, fc1_w, fc1_b.reshape(1, -1), fc2_w, fc2_b.reshape(1, -1), out_w, out_b.reshape(1, -1))


def _gat(x, src, dst, W, a_s, a_d, b, n):
    Hh, C = a_s.shape
    h = (x @ W).reshape(n, Hh, C)
    al_s = jnp.sum(h * a_s[None], axis=-1)
    al_d = jnp.sum(h * a_d[None], axis=-1)
    e = jax.nn.leaky_relu(al_s[src] + al_d[dst], 0.2)
    e = jnp.exp(e)
    s = jax.ops.segment_sum(e, dst, num_segments=n)
    a = e / s[dst]
    out = jax.ops.segment_sum(h[src] * a[:, :, None], dst, num_segments=n)
    return out.reshape(n, Hh * C) + b


def kernel(x, W1, a_src1, a_dst1, b1, W2, a_src2, a_dst2, b2, emb, conv_w, conv_b,
           fcg1_w, fcg1_b, fcxt1_w, fcxt1_b, fc1_w, fc1_b, fc2_w, fc2_b, out_w, out_b,
           edge_index, batch, target):
    n = x.shape[0]
    B = target.shape[0]
    loop = jnp.arange(n, dtype=edge_index.dtype)
    src = jnp.concatenate([edge_index[0], loop])
    dst = jnp.concatenate([edge_index[1], loop])
    x1 = jax.nn.elu(_gat(x, src, dst, W1, a_src1, a_dst1, b1, n))
    x2 = jax.nn.relu(_gat(x1, src, dst, W2, a_src2, a_dst2, b2, n))
    xg = jax.ops.segment_max(x2, batch, num_segments=B)
    xg = jax.nn.relu(xg @ fcg1_w + fcg1_b)
    et = emb[target]
    c = jax.lax.conv_general_dilated(et, conv_w, (1,), "VALID",
                                     dimension_numbers=("NCH", "OIH", "NCH"))
    c = jax.nn.relu(c + conv_b[None, :, None])
    xt = c.reshape(B, 32 * 121) @ fcxt1_w + fcxt1_b
    xc = jnp.concatenate([xg, xt], axis=1)
    return _mlp_head(xc, fc1_w, fc1_b, fc2_w, fc2_b, out_w, out_b)


# speedup vs baseline: 5.6027x; 5.2248x over previous
import functools

import jax
import jax.numpy as jnp
from jax import lax
from jax.experimental import pallas as pl
from jax.experimental.pallas import tpu as pltpu
from jax.experimental.pallas import tpu_sc as plsc

_N = 10000
_NPAD = 10240
_E2 = 170000
_EPAD = 180224
_CE = 512


def _edge_chunk(src_hbm, dst_hbm, table_hbm, als_v, ald_v, src_v, dst_v, w_v,
                rows_v, acc, sem, base_r, koff, ncol):
    pltpu.sync_copy(src_hbm.at[pl.ds(base_r, 4)], src_v)
    pltpu.sync_copy(dst_hbm.at[pl.ds(base_r, 4)], dst_v)

    for i in range(4):
        def wg(g, _, i=i):
            sl = pl.ds(g * 16, 16)
            s16 = src_v[i, sl]
            d16 = dst_v[i, sl]
            a = plsc.load_gather(als_v, [s16]) + plsc.load_gather(ald_v, [d16])
            z = jnp.where(a > 0.0, a, 0.2 * a)
            w_v[i, sl] = jnp.exp(z)
            src_v[i, sl] = s16 + koff
            return 0
        lax.fori_loop(0, 8, wg, 0)

    cps = [pltpu.async_copy(table_hbm.at[src_v.at[i]],
                            rows_v.at[pl.ds(i * 128, 128)], sem)
           for i in range(4)]
    for c in cps:
        c.wait()

    iotas = [lax.iota(jnp.int32, 16) + c * 16 for c in range(ncol // 16)]
    for i in range(4):
        def sg(g, _, i=i):
            w16 = w_v[i, pl.ds(g * 16, 16)]
            for j in range(16):
                e = i * 128 + g * 16 + j
                wb = jnp.broadcast_to(w16[j], (16,))
                row16 = jnp.broadcast_to(e, (16,))
                for col16 in iotas:
                    v = plsc.load_gather(rows_v, [row16, col16])
                    plsc.store_scatter(rows_v, [row16, col16], v * wb)
            return 0
        lax.fori_loop(0, 8, sg, 0)

    for i in range(4):
        pltpu.sync_copy(rows_v.at[pl.ds(i * 128, 128)],
                        acc.at[dst_v.at[i]], add=True)


def _gat1_sc(table_flat, als_flat, ald_flat, src2, dst2, zeros):
    mesh = plsc.VectorSubcoreMesh(core_axis_name="c", subcore_axis_name="s")

    @functools.partial(
        pl.kernel, mesh=mesh,
        compiler_params=pltpu.CompilerParams(
            needs_layout_passes=False, use_tc_tiling_on_sc=False),
        out_type=jax.ShapeDtypeStruct((10 * _NPAD, 80), jnp.float32),
        scratch_types=[
            pltpu.VMEM((4, 128), jnp.int32),
            pltpu.VMEM((4, 128), jnp.int32),
            pltpu.VMEM((4, 128), jnp.float32),
            pltpu.VMEM((_CE, 80), jnp.float32),
            pltpu.VMEM((_NPAD,), jnp.float32),
            pltpu.VMEM((_NPAD,), jnp.float32),
            pltpu.VMEM_SHARED((_N, 80), jnp.float32),
            pltpu.SemaphoreType.DMA,
        ],
    )
    def k(table_hbm, als_hbm, ald_hbm, src_hbm, dst_hbm, zeros_hbm, out_hbm,
          src_v, dst_v, w_v, rows_v, als_v, ald_v, acc, sem):
        cid = lax.axis_index("c")
        sid = lax.axis_index("s")
        row_lo = sid * 88

        def head(k_local, _):
            kh = cid * 5 + k_local
            koff = kh * _NPAD
            pltpu.sync_copy(zeros_hbm.at[pl.ds(sid * 625, 625)],
                            acc.at[pl.ds(sid * 625, 625)])
            pltpu.sync_copy(als_hbm.at[pl.ds(koff, _NPAD)], als_v)
            pltpu.sync_copy(ald_hbm.at[pl.ds(koff, _NPAD)], ald_v)
            plsc.subcore_barrier()

            def chunk(ci, _):
                _edge_chunk(src_hbm, dst_hbm, table_hbm, als_v, ald_v,
                            src_v, dst_v, w_v, rows_v, acc, sem,
                            row_lo + ci * 4, koff, 80)
                return 0
            lax.fori_loop(0, 22, chunk, 0)
            plsc.subcore_barrier()
            pltpu.sync_copy(acc.at[pl.ds(sid * 625, 625)],
                            out_hbm.at[pl.ds(koff + sid * 625, 625)])
            plsc.subcore_barrier()
            return 0

        lax.fori_loop(0, 5, head, 0)

    return k(table_flat, als_flat, ald_flat, src2, dst2, zeros)


def _gat2_sc(table, als, ald, src2, dst2, zeros, ncol):
    mesh = plsc.VectorSubcoreMesh(core_axis_name="c", subcore_axis_name="s")

    @functools.partial(
        pl.kernel, mesh=mesh,
        compiler_params=pltpu.CompilerParams(
            needs_layout_passes=False, use_tc_tiling_on_sc=False),
        out_type=jax.ShapeDtypeStruct((2 * _NPAD, ncol), jnp.float32),
        scratch_types=[
            pltpu.VMEM((4, 128), jnp.int32),
            pltpu.VMEM((4, 128), jnp.int32),
            pltpu.VMEM((4, 128), jnp.float32),
            pltpu.VMEM((_CE, ncol), jnp.float32),
            pltpu.VMEM((_NPAD,), jnp.float32),
            pltpu.VMEM((_NPAD,), jnp.float32),
            pltpu.VMEM_SHARED((_N, ncol), jnp.float32),
            pltpu.SemaphoreType.DMA,
        ],
    )
    def k(table_hbm, als_hbm, ald_hbm, src_hbm, dst_hbm, zeros_hbm, out_hbm,
          src_v, dst_v, w_v, rows_v, als_v, ald_v, acc, sem):
        cid = lax.axis_index("c")
        sid = lax.axis_index("s")
        wid = cid * 16 + sid
        row_lo = wid * 44
        pltpu.sync_copy(zeros_hbm.at[pl.ds(sid * 625, 625)],
                        acc.at[pl.ds(sid * 625, 625)])
        pltpu.sync_copy(als_hbm, als_v)
        pltpu.sync_copy(ald_hbm, ald_v)
        plsc.subcore_barrier()

        def chunk(ci, _):
            _edge_chunk(src_hbm, dst_hbm, table_hbm, als_v, ald_v,
                        src_v, dst_v, w_v, rows_v, acc, sem,
                        row_lo + ci * 4, 0, ncol)
            return 0
        lax.fori_loop(0, 11, chunk, 0)
        plsc.subcore_barrier()
        pltpu.sync_copy(acc.at[pl.ds(sid * 625, 625)],
                        out_hbm.at[pl.ds(cid * _NPAD + sid * 625, 625)])

    return k(table, als, ald, src2, dst2, zeros)


def _head_body(xc_ref, w1_ref, b1_ref, w2_ref, b2_ref, wo_ref, bo_ref, out_ref):
    xc = xc_ref[...]
    h = jnp.maximum(jnp.dot(xc, w1_ref[...], preferred_element_type=jnp.float32) + b1_ref[...], 0.0)
    h = jnp.maximum(jnp.dot(h, w2_ref[...], preferred_element_type=jnp.float32) + b2_ref[...], 0.0)
    out_ref[...] = jnp.dot(h, wo_ref[...], preferred_element_type=jnp.float32) + bo_ref[...]


def _mlp_head(xc, fc1_w, fc1_b, fc2_w, fc2_b, out_w, out_b):
    B = xc.shape[0]
    return pl.pallas_call(
        _head_body,
        out_shape=jax.ShapeDtypeStruct((B, 1), jnp.float32),
    )(xc, fc1_w, fc1_b.reshape(1, -1), fc2_w, fc2_b.reshape(1, -1), out_w, out_b.reshape(1, -1))


def kernel(x, W1, a_src1, a_dst1, b1, W2, a_src2, a_dst2, b2, emb, conv_w, conv_b,
           fcg1_w, fcg1_b, fcxt1_w, fcxt1_b, fc1_w, fc1_b, fc2_w, fc2_b, out_w, out_b,
           edge_index, batch, target):
    n = x.shape[0]
    B = target.shape[0]
    loop = jnp.arange(n, dtype=edge_index.dtype)
    npad_e = _EPAD - _E2
    src = jnp.concatenate([edge_index[0], loop,
                           jnp.full((npad_e,), n, jnp.int32)]).reshape(-1, 128)
    dst = jnp.concatenate([edge_index[1], loop,
                           jnp.zeros((npad_e,), jnp.int32)]).reshape(-1, 128)
    zeros80 = jnp.zeros((_NPAD, 80), jnp.float32)
    zeros64 = jnp.zeros((_NPAD, 64), jnp.float32)

    h1 = (x @ W1).reshape(n, 10, 78)
    als1 = jnp.einsum("nhc,hc->hn", h1, a_src1)
    ald1 = jnp.einsum("nhc,hc->hn", h1, a_dst1)
    h1t = jnp.transpose(h1, (1, 0, 2))
    table1 = jnp.concatenate([
        h1t, jnp.ones((10, n, 1), jnp.float32),
        jnp.zeros((10, n, 1), jnp.float32)], axis=-1)
    table1 = jnp.pad(table1, ((0, 0), (0, _NPAD - n), (0, 0))).reshape(10 * _NPAD, 80)
    als1p = jnp.pad(als1, ((0, 0), (0, _NPAD - n))).reshape(-1)
    ald1p = jnp.pad(ald1, ((0, 0), (0, _NPAD - n))).reshape(-1)
    num1 = _gat1_sc(table1, als1p, ald1p, src, dst, zeros80)
    num1 = num1.reshape(10, _NPAD, 80)[:, :n, :]
    x1 = jax.nn.elu(jnp.transpose(
        num1[:, :, :78] / (num1[:, :, 78:79] + 1e-16), (1, 0, 2)).reshape(n, 780) + b1)

    h2 = x1 @ W2
    als2 = jnp.pad(h2 @ a_src2[0], (0, _NPAD - n))
    ald2 = jnp.pad(h2 @ a_dst2[0], (0, _NPAD - n))
    t2a = jnp.pad(h2[:, :80], ((0, _NPAD - n), (0, 0)))
    t2b = jnp.concatenate([
        h2[:, 80:], jnp.ones((n, 1), jnp.float32),
        jnp.zeros((n, 15), jnp.float32)], axis=-1)
    t2b = jnp.pad(t2b, ((0, _NPAD - n), (0, 0)))
    outa = _gat2_sc(t2a, als2, ald2, src, dst, zeros80, 80)
    outb = _gat2_sc(t2b, als2, ald2, src, dst, zeros64, 64)
    numa = outa.reshape(2, _NPAD, 80).sum(0)[:n]
    numb = outb.reshape(2, _NPAD, 64).sum(0)[:n]
    den2 = numb[:, 48:49] + 1e-16
    x2 = jax.nn.relu(
        jnp.concatenate([numa, numb[:, :48]], axis=1) / den2 + b2)

    xg = jax.ops.segment_max(x2, batch, num_segments=B)
    xg = jax.nn.relu(xg @ fcg1_w + fcg1_b)
    et = emb[target]
    c = jax.lax.conv_general_dilated(et, conv_w, (1,), "VALID",
                                     dimension_numbers=("NCH", "OIH", "NCH"))
    c = jax.nn.relu(c + conv_b[None, :, None])
    xt = c.reshape(B, 32 * 121) @ fcxt1_w + fcxt1_b
    xc = jnp.concatenate([xg, xt], axis=1)
    return _mlp_head(xc, fc1_w, fc1_b, fc2_w, fc2_b, out_w, out_b)


# speedup vs baseline: 5.9235x; 1.0573x over previous
import functools

import jax
import jax.numpy as jnp
from jax import lax
from jax.experimental import pallas as pl
from jax.experimental.pallas import tpu as pltpu
from jax.experimental.pallas import tpu_sc as plsc

_N = 10000
_NPAD = 10240
_E2 = 170000
_EPAD = 180224
_CE = 512


def _edge_chunk(src_hbm, dst_hbm, table_hbm, als_v, ald_v, src_v, dst_v, w_v,
                rows_v, acc, sem, base_r, koff, ncol):
    pltpu.sync_copy(src_hbm.at[pl.ds(base_r, 4)], src_v)
    pltpu.sync_copy(dst_hbm.at[pl.ds(base_r, 4)], dst_v)

    for i in range(4):
        def wg(g, _, i=i):
            sl = pl.ds(g * 16, 16)
            s16 = src_v[i, sl]
            d16 = dst_v[i, sl]
            a = plsc.load_gather(als_v, [s16]) + plsc.load_gather(ald_v, [d16])
            z = jnp.where(a > 0.0, a, 0.2 * a)
            w_v[i, sl] = jnp.exp(z)
            src_v[i, sl] = s16 + koff
            return 0
        lax.fori_loop(0, 8, wg, 0)

    cps = [pltpu.async_copy(table_hbm.at[src_v.at[i]],
                            rows_v.at[pl.ds(i * 128, 128)], sem)
           for i in range(4)]
    for c in cps:
        c.wait()

    iotas = [lax.iota(jnp.int32, 16) + c * 16 for c in range(ncol // 16)]
    for i in range(4):
        def sg(g, _, i=i):
            w16 = w_v[i, pl.ds(g * 16, 16)]
            for j in range(16):
                e = i * 128 + g * 16 + j
                wb = jnp.broadcast_to(w16[j], (16,))
                row16 = jnp.broadcast_to(e, (16,))
                for col16 in iotas:
                    v = plsc.load_gather(rows_v, [row16, col16])
                    plsc.store_scatter(rows_v, [row16, col16], v * wb)
            return 0
        lax.fori_loop(0, 8, sg, 0)

    for i in range(4):
        pltpu.sync_copy(rows_v.at[pl.ds(i * 128, 128)],
                        acc.at[dst_v.at[i]], add=True)


def _gat1_sc(table_flat, als_flat, ald_flat, src2, dst2, zeros):
    mesh = plsc.VectorSubcoreMesh(core_axis_name="c", subcore_axis_name="s")

    @functools.partial(
        pl.kernel, mesh=mesh,
        compiler_params=pltpu.CompilerParams(
            needs_layout_passes=False, use_tc_tiling_on_sc=False),
        out_type=jax.ShapeDtypeStruct((10 * _NPAD, 80), jnp.float32),
        scratch_types=[
            pltpu.VMEM((4, 128), jnp.int32),
            pltpu.VMEM((4, 128), jnp.int32),
            pltpu.VMEM((4, 128), jnp.float32),
            pltpu.VMEM((_CE, 80), jnp.float32),
            pltpu.VMEM((_NPAD,), jnp.float32),
            pltpu.VMEM((_NPAD,), jnp.float32),
            pltpu.VMEM_SHARED((_N, 80), jnp.float32),
            pltpu.SemaphoreType.DMA,
        ],
    )
    def k(table_hbm, als_hbm, ald_hbm, src_hbm, dst_hbm, zeros_hbm, out_hbm,
          src_v, dst_v, w_v, rows_v, als_v, ald_v, acc, sem):
        cid = lax.axis_index("c")
        sid = lax.axis_index("s")
        row_lo = sid * 88

        def head(k_local, _):
            kh = cid * 5 + k_local
            koff = kh * _NPAD
            pltpu.sync_copy(zeros_hbm.at[pl.ds(sid * 625, 625)],
                            acc.at[pl.ds(sid * 625, 625)])
            pltpu.sync_copy(als_hbm.at[pl.ds(koff, _NPAD)], als_v)
            pltpu.sync_copy(ald_hbm.at[pl.ds(koff, _NPAD)], ald_v)
            plsc.subcore_barrier()

            def chunk(ci, _):
                _edge_chunk(src_hbm, dst_hbm, table_hbm, als_v, ald_v,
                            src_v, dst_v, w_v, rows_v, acc, sem,
                            row_lo + ci * 4, koff, 80)
                return 0
            lax.fori_loop(0, 22, chunk, 0)
            plsc.subcore_barrier()
            pltpu.sync_copy(acc.at[pl.ds(sid * 625, 625)],
                            out_hbm.at[pl.ds(koff + sid * 625, 625)])
            plsc.subcore_barrier()
            return 0

        lax.fori_loop(0, 5, head, 0)

    return k(table_flat, als_flat, ald_flat, src2, dst2, zeros)


def _gat2_sc(table, als, ald, src2, dst2, zeros, ncol):
    mesh = plsc.VectorSubcoreMesh(core_axis_name="c", subcore_axis_name="s")

    @functools.partial(
        pl.kernel, mesh=mesh,
        compiler_params=pltpu.CompilerParams(
            needs_layout_passes=False, use_tc_tiling_on_sc=False),
        out_type=jax.ShapeDtypeStruct((2 * _NPAD, ncol), jnp.float32),
        scratch_types=[
            pltpu.VMEM((4, 128), jnp.int32),
            pltpu.VMEM((4, 128), jnp.int32),
            pltpu.VMEM((4, 128), jnp.float32),
            pltpu.VMEM((_CE, ncol), jnp.float32),
            pltpu.VMEM((_NPAD,), jnp.float32),
            pltpu.VMEM((_NPAD,), jnp.float32),
            pltpu.VMEM_SHARED((_N, ncol), jnp.float32),
            pltpu.SemaphoreType.DMA,
        ],
    )
    def k(table_hbm, als_hbm, ald_hbm, src_hbm, dst_hbm, zeros_hbm, out_hbm,
          src_v, dst_v, w_v, rows_v, als_v, ald_v, acc, sem):
        cid = lax.axis_index("c")
        sid = lax.axis_index("s")
        wid = cid * 16 + sid
        row_lo = wid * 44
        pltpu.sync_copy(zeros_hbm.at[pl.ds(sid * 625, 625)],
                        acc.at[pl.ds(sid * 625, 625)])
        pltpu.sync_copy(als_hbm, als_v)
        pltpu.sync_copy(ald_hbm, ald_v)
        plsc.subcore_barrier()

        def chunk(ci, _):
            _edge_chunk(src_hbm, dst_hbm, table_hbm, als_v, ald_v,
                        src_v, dst_v, w_v, rows_v, acc, sem,
                        row_lo + ci * 4, 0, ncol)
            return 0
        lax.fori_loop(0, 11, chunk, 0)
        plsc.subcore_barrier()
        pltpu.sync_copy(acc.at[pl.ds(sid * 625, 625)],
                        out_hbm.at[pl.ds(cid * _NPAD + sid * 625, 625)])

    return k(table, als, ald, src2, dst2, zeros)


def _pre1_body(x_ref, w1_ref, as_ref, ad_ref, t_ref, als_ref, ald_ref):
    i = pl.program_id(0)
    xb = x_ref[...]
    h = jnp.dot(xb, w1_ref[...], preferred_element_type=jnp.float32)
    h3 = h.reshape(1280, 10, 78)
    als = jnp.sum(h3 * as_ref[...][None], axis=-1)
    ald = jnp.sum(h3 * ad_ref[...][None], axis=-1)
    als_ref[...] = als.T
    ald_ref[...] = ald.T
    rows = i * 1280 + jax.lax.broadcasted_iota(jnp.int32, (1280, 1), 0)
    ones = jnp.where(rows < 10000, 1.0, 0.0)[None, :, :]
    h3t = jnp.transpose(h3, (1, 0, 2))
    t_ref[...] = jnp.concatenate(
        [h3t, jnp.broadcast_to(ones, (10, 1280, 1)),
         jnp.zeros((10, 1280, 1), jnp.float32)], axis=-1)


def _pre1(x_pad, W1, a_src1, a_dst1):
    return pl.pallas_call(
        _pre1_body,
        grid=(8,),
        in_specs=[
            pl.BlockSpec((1280, 78), lambda i: (i, 0)),
            pl.BlockSpec((78, 780), lambda i: (0, 0)),
            pl.BlockSpec((10, 78), lambda i: (0, 0)),
            pl.BlockSpec((10, 78), lambda i: (0, 0)),
        ],
        out_specs=[
            pl.BlockSpec((10, 1280, 80), lambda i: (0, i, 0)),
            pl.BlockSpec((10, 1280), lambda i: (0, i)),
            pl.BlockSpec((10, 1280), lambda i: (0, i)),
        ],
        out_shape=[
            jax.ShapeDtypeStruct((10, _NPAD, 80), jnp.float32),
            jax.ShapeDtypeStruct((10, _NPAD), jnp.float32),
            jax.ShapeDtypeStruct((10, _NPAD), jnp.float32),
        ],
    )(x_pad, W1, a_src1, a_dst1)


def _mid_body(num_ref, b1_ref, w2_ref, as2_ref, ad2_ref, t2a_ref, t2b_ref, al2_ref):
    i = pl.program_id(0)
    num = num_ref[...]
    x1 = num[:, :, :78] / (num[:, :, 78:79] + 1e-16)
    x1 = jnp.transpose(x1, (1, 0, 2)).reshape(1280, 780) + b1_ref[...]
    x1 = jnp.where(x1 > 0, x1, jnp.exp(jnp.minimum(x1, 0.0)) - 1.0)
    rows = i * 1280 + jax.lax.broadcasted_iota(jnp.int32, (1280, 1), 0)
    valid = rows < 10000
    x1 = jnp.where(valid, x1, 0.0)
    h2 = jnp.dot(x1, w2_ref[...], preferred_element_type=jnp.float32)
    t2a_ref[...] = h2[:, :80]
    ones = jnp.where(valid, 1.0, 0.0)
    t2b_ref[...] = jnp.concatenate(
        [h2[:, 80:], ones, jnp.zeros((1280, 15), jnp.float32)], axis=-1)
    al2_ref[...] = jnp.concatenate(
        [jnp.dot(h2, as2_ref[...].T, preferred_element_type=jnp.float32),
         jnp.dot(h2, ad2_ref[...].T, preferred_element_type=jnp.float32)], axis=-1)


def _mid(num1, b1, W2, a_src2, a_dst2):
    return pl.pallas_call(
        _mid_body,
        grid=(8,),
        in_specs=[
            pl.BlockSpec((10, 1280, 80), lambda i: (0, i, 0)),
            pl.BlockSpec((1, 780), lambda i: (0, 0)),
            pl.BlockSpec((780, 128), lambda i: (0, 0)),
            pl.BlockSpec((1, 128), lambda i: (0, 0)),
            pl.BlockSpec((1, 128), lambda i: (0, 0)),
        ],
        out_specs=[
            pl.BlockSpec((1280, 80), lambda i: (i, 0)),
            pl.BlockSpec((1280, 64), lambda i: (i, 0)),
            pl.BlockSpec((1280, 2), lambda i: (i, 0)),
        ],
        out_shape=[
            jax.ShapeDtypeStruct((_NPAD, 80), jnp.float32),
            jax.ShapeDtypeStruct((_NPAD, 64), jnp.float32),
            jax.ShapeDtypeStruct((_NPAD, 2), jnp.float32),
        ],
    )(num1.reshape(10, _NPAD, 80), b1.reshape(1, 780), W2, a_src2, a_dst2)


def _post2_body(oa_ref, ob_ref, b2_ref, x2_ref):
    i = pl.program_id(0)
    na = oa_ref[0] + oa_ref[1]
    nb = ob_ref[0] + ob_ref[1]
    den = nb[:, 48:49] + 1e-16
    x2 = jnp.concatenate([na, nb[:, :48]], axis=-1) / den + b2_ref[...]
    x2 = jnp.maximum(x2, 0.0)
    rows = i * 1280 + jax.lax.broadcasted_iota(jnp.int32, (1280, 1), 0)
    x2_ref[...] = jnp.where(rows < 10000, x2, -jnp.inf)


def _post2(outa, outb, b2):
    return pl.pallas_call(
        _post2_body,
        grid=(8,),
        in_specs=[
            pl.BlockSpec((2, 1280, 80), lambda i: (0, i, 0)),
            pl.BlockSpec((2, 1280, 64), lambda i: (0, i, 0)),
            pl.BlockSpec((1, 128), lambda i: (0, 0)),
        ],
        out_specs=pl.BlockSpec((1280, 128), lambda i: (i, 0)),
        out_shape=jax.ShapeDtypeStruct((_NPAD, 128), jnp.float32),
    )(outa.reshape(2, _NPAD, 80), outb.reshape(2, _NPAD, 64), b2.reshape(1, 128))


def _conv_body(t_ref, emb_ref, w2d_ref, cb_ref, c_ref):
    t = t_ref[...].reshape(1000, 1)
    oh = (jax.lax.broadcasted_iota(jnp.int32, (1000, 128), 1) == t).astype(jnp.float32)
    a = jnp.dot(oh, emb_ref[...], preferred_element_type=jnp.float32)
    p = jnp.dot(w2d_ref[...], a, preferred_element_type=jnp.float32)
    p3 = p.reshape(32, 8, 128)
    c = p3[:, 0, 0:121]
    for k in range(1, 8):
        c = c + p3[:, k, k:k + 121]
    c_ref[...] = jnp.maximum(c + cb_ref[...], 0.0).reshape(1, 32, 121)


def _conv(target3, emb_pad, w2d, conv_b):
    return pl.pallas_call(
        _conv_body,
        grid=(128,),
        in_specs=[
            pl.BlockSpec((1, 1, 1000), lambda i: (i, 0, 0)),
            pl.BlockSpec((128, 128), lambda i: (0, 0)),
            pl.BlockSpec((256, 1000), lambda i: (0, 0)),
            pl.BlockSpec((32, 1), lambda i: (0, 0)),
        ],
        out_specs=pl.BlockSpec((1, 32, 121), lambda i: (i, 0, 0)),
        out_shape=jax.ShapeDtypeStruct((128, 32, 121), jnp.float32),
    )(target3, emb_pad, w2d, conv_b.reshape(32, 1))


def _head_body(xm_ref, cf_ref, g_w, g_b, xt_w, xt_b, w1_ref, b1_ref,
               w2_ref, b2_ref, wo_ref, bo_ref, out_ref):
    xg = jnp.maximum(jnp.dot(xm_ref[...], g_w[...], preferred_element_type=jnp.float32) + g_b[...], 0.0)
    xt = jnp.dot(cf_ref[...], xt_w[...], preferred_element_type=jnp.float32) + xt_b[...]
    xc = jnp.concatenate([xg, xt], axis=1)
    h = jnp.maximum(jnp.dot(xc, w1_ref[...], preferred_element_type=jnp.float32) + b1_ref[...], 0.0)
    h = jnp.maximum(jnp.dot(h, w2_ref[...], preferred_element_type=jnp.float32) + b2_ref[...], 0.0)
    out_ref[...] = jnp.dot(h, wo_ref[...], preferred_element_type=jnp.float32) + bo_ref[...]


def _mlp_head(xmax, c_flat, fcg1_w, fcg1_b, fcxt1_w, fcxt1_b,
              fc1_w, fc1_b, fc2_w, fc2_b, out_w, out_b):
    B = xmax.shape[0]
    return pl.pallas_call(
        _head_body,
        out_shape=jax.ShapeDtypeStruct((B, 1), jnp.float32),
    )(xmax, c_flat, fcg1_w, fcg1_b.reshape(1, -1), fcxt1_w, fcxt1_b.reshape(1, -1),
      fc1_w, fc1_b.reshape(1, -1), fc2_w, fc2_b.reshape(1, -1), out_w, out_b.reshape(1, -1))


def kernel(x, W1, a_src1, a_dst1, b1, W2, a_src2, a_dst2, b2, emb, conv_w, conv_b,
           fcg1_w, fcg1_b, fcxt1_w, fcxt1_b, fc1_w, fc1_b, fc2_w, fc2_b, out_w, out_b,
           edge_index, batch, target):
    n = x.shape[0]
    B = target.shape[0]
    loop = jnp.arange(n, dtype=edge_index.dtype)
    npad_e = _EPAD - _E2
    src = jnp.concatenate([edge_index[0], loop,
                           jnp.full((npad_e,), n, jnp.int32)]).reshape(-1, 128)
    dst = jnp.concatenate([edge_index[1], loop,
                           jnp.zeros((npad_e,), jnp.int32)]).reshape(-1, 128)
    zeros80 = jnp.zeros((_NPAD, 80), jnp.float32)
    zeros64 = jnp.zeros((_NPAD, 64), jnp.float32)

    x_pad = jnp.pad(x, ((0, _NPAD - n), (0, 0)))
    table1, als1, ald1 = _pre1(x_pad, W1, a_src1, a_dst1)
    num1 = _gat1_sc(table1.reshape(10 * _NPAD, 80), als1.reshape(-1),
                    ald1.reshape(-1), src, dst, zeros80)

    t2a, t2b, al2 = _mid(num1, b1, W2, a_src2, a_dst2)

    als2 = al2[:, 0]
    ald2 = al2[:, 1]
    outa = _gat2_sc(t2a, als2, ald2, src, dst, zeros80, 80)
    outb = _gat2_sc(t2b, als2, ald2, src, dst, zeros64, 64)
    x2 = _post2(outa, outb, b2)

    xmax = jax.ops.segment_max(x2[:n], batch, num_segments=B)

    w2d = conv_w.transpose(0, 2, 1).reshape(256, 1000)
    emb_pad = jnp.pad(emb, ((0, 128 - emb.shape[0]), (0, 0)))
    c = _conv(target.reshape(B, 1, 1000), emb_pad, w2d, conv_b)

    return _mlp_head(xmax, c.reshape(B, 32 * 121), fcg1_w, fcg1_b,
                     fcxt1_w, fcxt1_b, fc1_w, fc1_b, fc2_w, fc2_b, out_w, out_b)


# speedup vs baseline: 7.8087x; 1.3183x over previous
import functools

import jax
import jax.numpy as jnp
from jax import lax
from jax.experimental import pallas as pl
from jax.experimental.pallas import tpu as pltpu
from jax.experimental.pallas import tpu_sc as plsc

_N = 10000
_NPAD = 10240
_E2 = 170000
_EPAD = 180224
_CE = 512


def _edge_chunk(src_hbm, dst_hbm, table_hbm, als_v, ald_v, src_v, dst_v, w_v,
                rows_v, acc, sem, base_r, koff, ncol):
    pltpu.sync_copy(src_hbm.at[pl.ds(base_r, 4)], src_v)
    pltpu.sync_copy(dst_hbm.at[pl.ds(base_r, 4)], dst_v)

    for i in range(4):
        def wg(g, _, i=i):
            sl = pl.ds(g * 16, 16)
            s16 = src_v[i, sl]
            d16 = dst_v[i, sl]
            a = plsc.load_gather(als_v, [s16]) + plsc.load_gather(ald_v, [d16])
            z = jnp.where(a > 0.0, a, 0.2 * a)
            w_v[i, sl] = jnp.exp(z)
            src_v[i, sl] = s16 + koff
            return 0
        lax.fori_loop(0, 8, wg, 0)

    cps = [pltpu.async_copy(table_hbm.at[src_v.at[i]],
                            rows_v.at[pl.ds(i * 128, 128)], sem)
           for i in range(4)]
    for c in cps:
        c.wait()

    for i in range(4):
        def sg(g, _, i=i):
            w16 = w_v[i, pl.ds(g * 16, 16)]
            for j in range(16):
                e = i * 128 + g * 16 + j
                wb = jnp.broadcast_to(w16[j], (16,))
                for c in range(ncol // 16):
                    sl = pl.ds(c * 16, 16)
                    rows_v[e, sl] = rows_v[e, sl] * wb
            return 0
        lax.fori_loop(0, 8, sg, 0)

    for i in range(4):
        pltpu.sync_copy(rows_v.at[pl.ds(i * 128, 128)],
                        acc.at[dst_v.at[i]], add=True)


def _gat1_sc(table_flat, als_flat, ald_flat, src2, dst2, zeros):
    mesh = plsc.VectorSubcoreMesh(core_axis_name="c", subcore_axis_name="s")

    @functools.partial(
        pl.kernel, mesh=mesh,
        compiler_params=pltpu.CompilerParams(
            needs_layout_passes=False, use_tc_tiling_on_sc=False),
        out_type=jax.ShapeDtypeStruct((10 * _NPAD, 80), jnp.float32),
        scratch_types=[
            pltpu.VMEM((4, 128), jnp.int32),
            pltpu.VMEM((4, 128), jnp.int32),
            pltpu.VMEM((4, 128), jnp.float32),
            pltpu.VMEM((_CE, 80), jnp.float32),
            pltpu.VMEM((_NPAD,), jnp.float32),
            pltpu.VMEM((_NPAD,), jnp.float32),
            pltpu.VMEM_SHARED((_N, 80), jnp.float32),
            pltpu.SemaphoreType.DMA,
        ],
    )
    def k(table_hbm, als_hbm, ald_hbm, src_hbm, dst_hbm, zeros_hbm, out_hbm,
          src_v, dst_v, w_v, rows_v, als_v, ald_v, acc, sem):
        cid = lax.axis_index("c")
        sid = lax.axis_index("s")
        row_lo = sid * 88

        def head(k_local, _):
            kh = cid * 5 + k_local
            koff = kh * _NPAD
            pltpu.sync_copy(zeros_hbm.at[pl.ds(sid * 625, 625)],
                            acc.at[pl.ds(sid * 625, 625)])
            pltpu.sync_copy(als_hbm.at[pl.ds(koff, _NPAD)], als_v)
            pltpu.sync_copy(ald_hbm.at[pl.ds(koff, _NPAD)], ald_v)
            plsc.subcore_barrier()

            def chunk(ci, _):
                _edge_chunk(src_hbm, dst_hbm, table_hbm, als_v, ald_v,
                            src_v, dst_v, w_v, rows_v, acc, sem,
                            row_lo + ci * 4, koff, 80)
                return 0
            lax.fori_loop(0, 22, chunk, 0)
            plsc.subcore_barrier()
            pltpu.sync_copy(acc.at[pl.ds(sid * 625, 625)],
                            out_hbm.at[pl.ds(koff + sid * 625, 625)])
            plsc.subcore_barrier()
            return 0

        lax.fori_loop(0, 5, head, 0)

    return k(table_flat, als_flat, ald_flat, src2, dst2, zeros)


def _gat2_sc(table, als, ald, src2, dst2, zeros, ncol):
    mesh = plsc.VectorSubcoreMesh(core_axis_name="c", subcore_axis_name="s")

    @functools.partial(
        pl.kernel, mesh=mesh,
        compiler_params=pltpu.CompilerParams(
            needs_layout_passes=False, use_tc_tiling_on_sc=False),
        out_type=jax.ShapeDtypeStruct((2 * _NPAD, ncol), jnp.float32),
        scratch_types=[
            pltpu.VMEM((4, 128), jnp.int32),
            pltpu.VMEM((4, 128), jnp.int32),
            pltpu.VMEM((4, 128), jnp.float32),
            pltpu.VMEM((_CE, ncol), jnp.float32),
            pltpu.VMEM((_NPAD,), jnp.float32),
            pltpu.VMEM((_NPAD,), jnp.float32),
            pltpu.VMEM_SHARED((_N, ncol), jnp.float32),
            pltpu.SemaphoreType.DMA,
        ],
    )
    def k(table_hbm, als_hbm, ald_hbm, src_hbm, dst_hbm, zeros_hbm, out_hbm,
          src_v, dst_v, w_v, rows_v, als_v, ald_v, acc, sem):
        cid = lax.axis_index("c")
        sid = lax.axis_index("s")
        wid = cid * 16 + sid
        row_lo = wid * 44
        pltpu.sync_copy(zeros_hbm.at[pl.ds(sid * 625, 625)],
                        acc.at[pl.ds(sid * 625, 625)])
        pltpu.sync_copy(als_hbm, als_v)
        pltpu.sync_copy(ald_hbm, ald_v)
        plsc.subcore_barrier()

        def chunk(ci, _):
            _edge_chunk(src_hbm, dst_hbm, table_hbm, als_v, ald_v,
                        src_v, dst_v, w_v, rows_v, acc, sem,
                        row_lo + ci * 4, 0, ncol)
            return 0
        lax.fori_loop(0, 11, chunk, 0)
        plsc.subcore_barrier()
        pltpu.sync_copy(acc.at[pl.ds(sid * 625, 625)],
                        out_hbm.at[pl.ds(cid * _NPAD + sid * 625, 625)])

    return k(table, als, ald, src2, dst2, zeros)


def _pre1_body(x_ref, w1_ref, as_ref, ad_ref, t_ref, als_ref, ald_ref):
    i = pl.program_id(0)
    xb = x_ref[...]
    h = jnp.dot(xb, w1_ref[...], preferred_element_type=jnp.float32)
    h3 = h.reshape(1280, 10, 78)
    als = jnp.sum(h3 * as_ref[...][None], axis=-1)
    ald = jnp.sum(h3 * ad_ref[...][None], axis=-1)
    als_ref[...] = als.T
    ald_ref[...] = ald.T
    rows = i * 1280 + jax.lax.broadcasted_iota(jnp.int32, (1280, 1), 0)
    ones = jnp.where(rows < 10000, 1.0, 0.0)[None, :, :]
    h3t = jnp.transpose(h3, (1, 0, 2))
    t_ref[...] = jnp.concatenate(
        [h3t, jnp.broadcast_to(ones, (10, 1280, 1)),
         jnp.zeros((10, 1280, 1), jnp.float32)], axis=-1)


def _pre1(x_pad, W1, a_src1, a_dst1):
    return pl.pallas_call(
        _pre1_body,
        grid=(8,),
        in_specs=[
            pl.BlockSpec((1280, 78), lambda i: (i, 0)),
            pl.BlockSpec((78, 780), lambda i: (0, 0)),
            pl.BlockSpec((10, 78), lambda i: (0, 0)),
            pl.BlockSpec((10, 78), lambda i: (0, 0)),
        ],
        out_specs=[
            pl.BlockSpec((10, 1280, 80), lambda i: (0, i, 0)),
            pl.BlockSpec((10, 1280), lambda i: (0, i)),
            pl.BlockSpec((10, 1280), lambda i: (0, i)),
        ],
        out_shape=[
            jax.ShapeDtypeStruct((10, _NPAD, 80), jnp.float32),
            jax.ShapeDtypeStruct((10, _NPAD), jnp.float32),
            jax.ShapeDtypeStruct((10, _NPAD), jnp.float32),
        ],
    )(x_pad, W1, a_src1, a_dst1)


def _mid_body(num_ref, b1_ref, w2_ref, as2_ref, ad2_ref, t2a_ref, t2b_ref, al2_ref):
    i = pl.program_id(0)
    num = num_ref[...]
    x1 = num[:, :, :78] / (num[:, :, 78:79] + 1e-16)
    x1 = jnp.transpose(x1, (1, 0, 2)).reshape(1280, 780) + b1_ref[...]
    x1 = jnp.where(x1 > 0, x1, jnp.exp(jnp.minimum(x1, 0.0)) - 1.0)
    rows = i * 1280 + jax.lax.broadcasted_iota(jnp.int32, (1280, 1), 0)
    valid = rows < 10000
    x1 = jnp.where(valid, x1, 0.0)
    h2 = jnp.dot(x1, w2_ref[...], preferred_element_type=jnp.float32)
    t2a_ref[...] = h2[:, :80]
    ones = jnp.where(valid, 1.0, 0.0)
    t2b_ref[...] = jnp.concatenate(
        [h2[:, 80:], ones, jnp.zeros((1280, 15), jnp.float32)], axis=-1)
    al2_ref[...] = jnp.concatenate(
        [jnp.dot(h2, as2_ref[...].T, preferred_element_type=jnp.float32),
         jnp.dot(h2, ad2_ref[...].T, preferred_element_type=jnp.float32)], axis=-1)


def _mid(num1, b1, W2, a_src2, a_dst2):
    return pl.pallas_call(
        _mid_body,
        grid=(8,),
        in_specs=[
            pl.BlockSpec((10, 1280, 80), lambda i: (0, i, 0)),
            pl.BlockSpec((1, 780), lambda i: (0, 0)),
            pl.BlockSpec((780, 128), lambda i: (0, 0)),
            pl.BlockSpec((1, 128), lambda i: (0, 0)),
            pl.BlockSpec((1, 128), lambda i: (0, 0)),
        ],
        out_specs=[
            pl.BlockSpec((1280, 80), lambda i: (i, 0)),
            pl.BlockSpec((1280, 64), lambda i: (i, 0)),
            pl.BlockSpec((1280, 2), lambda i: (i, 0)),
        ],
        out_shape=[
            jax.ShapeDtypeStruct((_NPAD, 80), jnp.float32),
            jax.ShapeDtypeStruct((_NPAD, 64), jnp.float32),
            jax.ShapeDtypeStruct((_NPAD, 2), jnp.float32),
        ],
    )(num1.reshape(10, _NPAD, 80), b1.reshape(1, 780), W2, a_src2, a_dst2)


def _post2_body(oa_ref, ob_ref, b2_ref, x2_ref):
    i = pl.program_id(0)
    na = oa_ref[0] + oa_ref[1]
    nb = ob_ref[0] + ob_ref[1]
    den = nb[:, 48:49] + 1e-16
    x2 = jnp.concatenate([na, nb[:, :48]], axis=-1) / den + b2_ref[...]
    x2 = jnp.maximum(x2, 0.0)
    rows = i * 1280 + jax.lax.broadcasted_iota(jnp.int32, (1280, 1), 0)
    x2_ref[...] = jnp.where(rows < 10000, x2, -jnp.inf)


def _post2(outa, outb, b2):
    return pl.pallas_call(
        _post2_body,
        grid=(8,),
        in_specs=[
            pl.BlockSpec((2, 1280, 80), lambda i: (0, i, 0)),
            pl.BlockSpec((2, 1280, 64), lambda i: (0, i, 0)),
            pl.BlockSpec((1, 128), lambda i: (0, 0)),
        ],
        out_specs=pl.BlockSpec((1280, 128), lambda i: (i, 0)),
        out_shape=jax.ShapeDtypeStruct((_NPAD, 128), jnp.float32),
    )(outa.reshape(2, _NPAD, 80), outb.reshape(2, _NPAD, 64), b2.reshape(1, 128))


def _conv_body(t_ref, emb_ref, w2d_ref, cb_ref, c_ref):
    t = t_ref[...].reshape(1000, 1)
    oh = (jax.lax.broadcasted_iota(jnp.int32, (1000, 128), 1) == t).astype(jnp.float32)
    a = jnp.dot(oh, emb_ref[...], preferred_element_type=jnp.float32)
    p = jnp.dot(w2d_ref[...], a, preferred_element_type=jnp.float32)
    p3 = p.reshape(32, 8, 128)
    c = p3[:, 0, 0:121]
    for k in range(1, 8):
        c = c + p3[:, k, k:k + 121]
    c_ref[...] = jnp.maximum(c + cb_ref[...], 0.0).reshape(1, 32, 121)


def _conv(target3, emb_pad, w2d, conv_b):
    return pl.pallas_call(
        _conv_body,
        grid=(128,),
        in_specs=[
            pl.BlockSpec((1, 1, 1000), lambda i: (i, 0, 0)),
            pl.BlockSpec((128, 128), lambda i: (0, 0)),
            pl.BlockSpec((256, 1000), lambda i: (0, 0)),
            pl.BlockSpec((32, 1), lambda i: (0, 0)),
        ],
        out_specs=pl.BlockSpec((1, 32, 121), lambda i: (i, 0, 0)),
        out_shape=jax.ShapeDtypeStruct((128, 32, 121), jnp.float32),
    )(target3, emb_pad, w2d, conv_b.reshape(32, 1))


def _head_body(xm_ref, cf_ref, g_w, g_b, xt_w, xt_b, w1_ref, b1_ref,
               w2_ref, b2_ref, wo_ref, bo_ref, out_ref):
    xg = jnp.maximum(jnp.dot(xm_ref[...], g_w[...], preferred_element_type=jnp.float32) + g_b[...], 0.0)
    xt = jnp.dot(cf_ref[...], xt_w[...], preferred_element_type=jnp.float32) + xt_b[...]
    xc = jnp.concatenate([xg, xt], axis=1)
    h = jnp.maximum(jnp.dot(xc, w1_ref[...], preferred_element_type=jnp.float32) + b1_ref[...], 0.0)
    h = jnp.maximum(jnp.dot(h, w2_ref[...], preferred_element_type=jnp.float32) + b2_ref[...], 0.0)
    out_ref[...] = jnp.dot(h, wo_ref[...], preferred_element_type=jnp.float32) + bo_ref[...]


def _mlp_head(xmax, c_flat, fcg1_w, fcg1_b, fcxt1_w, fcxt1_b,
              fc1_w, fc1_b, fc2_w, fc2_b, out_w, out_b):
    B = xmax.shape[0]
    return pl.pallas_call(
        _head_body,
        out_shape=jax.ShapeDtypeStruct((B, 1), jnp.float32),
    )(xmax, c_flat, fcg1_w, fcg1_b.reshape(1, -1), fcxt1_w, fcxt1_b.reshape(1, -1),
      fc1_w, fc1_b.reshape(1, -1), fc2_w, fc2_b.reshape(1, -1), out_w, out_b.reshape(1, -1))


def kernel(x, W1, a_src1, a_dst1, b1, W2, a_src2, a_dst2, b2, emb, conv_w, conv_b,
           fcg1_w, fcg1_b, fcxt1_w, fcxt1_b, fc1_w, fc1_b, fc2_w, fc2_b, out_w, out_b,
           edge_index, batch, target):
    n = x.shape[0]
    B = target.shape[0]
    loop = jnp.arange(n, dtype=edge_index.dtype)
    npad_e = _EPAD - _E2
    src = jnp.concatenate([edge_index[0], loop,
                           jnp.full((npad_e,), n, jnp.int32)]).reshape(-1, 128)
    dst = jnp.concatenate([edge_index[1], loop,
                           jnp.zeros((npad_e,), jnp.int32)]).reshape(-1, 128)
    zeros80 = jnp.zeros((_NPAD, 80), jnp.float32)
    zeros64 = jnp.zeros((_NPAD, 64), jnp.float32)

    x_pad = jnp.pad(x, ((0, _NPAD - n), (0, 0)))
    table1, als1, ald1 = _pre1(x_pad, W1, a_src1, a_dst1)
    num1 = _gat1_sc(table1.reshape(10 * _NPAD, 80), als1.reshape(-1),
                    ald1.reshape(-1), src, dst, zeros80)

    t2a, t2b, al2 = _mid(num1, b1, W2, a_src2, a_dst2)

    als2 = al2[:, 0]
    ald2 = al2[:, 1]
    outa = _gat2_sc(t2a, als2, ald2, src, dst, zeros80, 80)
    outb = _gat2_sc(t2b, als2, ald2, src, dst, zeros64, 64)
    x2 = _post2(outa, outb, b2)

    xmax = jax.ops.segment_max(x2[:n], batch, num_segments=B)

    w2d = conv_w.transpose(0, 2, 1).reshape(256, 1000)
    emb_pad = jnp.pad(emb, ((0, 128 - emb.shape[0]), (0, 0)))
    c = _conv(target.reshape(B, 1, 1000), emb_pad, w2d, conv_b)

    return _mlp_head(xmax, c.reshape(B, 32 * 121), fcg1_w, fcg1_b,
                     fcxt1_w, fcxt1_b, fc1_w, fc1_b, fc2_w, fc2_b, out_w, out_b)
